# Initial kernel scaffold; baseline (speedup 1.0000x reference)
import jax
import jax.numpy as jnp
from jax.experimental import pallas as pl


def kernel(x, table):
    raise NotImplementedError("write your pallas kernel here")


# speedup vs baseline: 3.6013x; 3.6013x over previous
import functools

import jax
import jax.numpy as jnp
from jax import lax
from jax.experimental import pallas as pl
from jax.experimental.pallas import tpu as pltpu
from jax.experimental.pallas import tpu_sc as plsc

_INFO = plsc.get_sparse_core_info()
_NC = _INFO.num_cores
_NS = _INFO.num_subcores
_NW = _NC * _NS

_CHUNK = 128
_DPAD = 128


def _make_gather(n_chunks: int, d: int):
    mesh = plsc.VectorSubcoreMesh(core_axis_name="c", subcore_axis_name="s")
    b_per_w = n_chunks * _CHUNK
    total = _NW * b_per_w

    @functools.partial(
        pl.kernel,
        mesh=mesh,
        out_type=jax.ShapeDtypeStruct((total, _DPAD), jnp.float32),
        scratch_types=[
            pltpu.VMEM((n_chunks, _CHUNK), jnp.int32),
            pltpu.VMEM((_CHUNK, _DPAD), jnp.float32),
            pltpu.SemaphoreType.DMA,
        ],
    )
    def gather_kernel(idx_hbm, table_hbm, out_hbm, idx_v, rows_v, gsem):
        wid = lax.axis_index("s") * _NC + lax.axis_index("c")
        pltpu.sync_copy(idx_hbm.at[wid], idx_v)
        base = wid * b_per_w

        def body(j, carry):
            pltpu.async_copy(table_hbm.at[idx_v.at[j]], rows_v, gsem).wait()
            pltpu.sync_copy(rows_v, out_hbm.at[pl.ds(base + j * _CHUNK, _CHUNK)])
            return carry

        lax.fori_loop(0, n_chunks, body, 0)

    return gather_kernel


def kernel(x, table):
    b = x.shape[0] * x.shape[1]
    d = table.shape[1]
    n_chunks = b // (_NW * _CHUNK)
    idx = jnp.reshape(x.astype(jnp.int32), (_NW, n_chunks, _CHUNK))
    tpad = jnp.pad(table, ((0, 0), (0, _DPAD - d)))
    out = _make_gather(n_chunks, d)(idx, tpad)
    return jnp.reshape(out[:, :d], (x.shape[0], x.shape[1], d))


# speedup vs baseline: 4.2136x; 1.1700x over previous
import functools

import jax
import jax.numpy as jnp
from jax import lax
from jax.experimental import pallas as pl
from jax.experimental.pallas import tpu as pltpu
from jax.experimental.pallas import tpu_sc as plsc

_INFO = plsc.get_sparse_core_info()
_NC = _INFO.num_cores
_NS = _INFO.num_subcores
_NW = _NC * _NS

_CHUNK = 128
_DPAD = 128
_NBUF = 4


def _make_gather(n_chunks: int, d: int):
    mesh = plsc.VectorSubcoreMesh(core_axis_name="c", subcore_axis_name="s")
    b_per_w = n_chunks * _CHUNK
    total = _NW * b_per_w
    n_grp = n_chunks // _NBUF

    @functools.partial(
        pl.kernel,
        mesh=mesh,
        out_type=jax.ShapeDtypeStruct((total, _DPAD), jnp.float32),
        scratch_types=[
            pltpu.VMEM((n_chunks, _CHUNK), jnp.int32),
            [pltpu.VMEM((_CHUNK, _DPAD), jnp.float32) for _ in range(_NBUF)],
            [pltpu.SemaphoreType.DMA for _ in range(_NBUF)],
            [pltpu.SemaphoreType.DMA for _ in range(_NBUF)],
        ],
    )
    def gather_kernel(idx_hbm, table_hbm, out_hbm, idx_v, rows, gsems, ssems):
        wid = lax.axis_index("s") * _NC + lax.axis_index("c")
        pltpu.sync_copy(idx_hbm.at[wid], idx_v)
        base = wid * b_per_w

        def body(k, carry):
            c0 = k * _NBUF

            @pl.when(k > 0)
            def _drain_prev():
                for i in range(_NBUF):
                    pltpu.make_async_copy(
                        rows[i], out_hbm.at[pl.ds(base, _CHUNK)], ssems[i]
                    ).wait()

            gathers = [
                pltpu.async_copy(
                    table_hbm.at[idx_v.at[c0 + i]], rows[i], gsems[i]
                )
                for i in range(_NBUF)
            ]
            for i in range(_NBUF):
                gathers[i].wait()
                pltpu.async_copy(
                    rows[i],
                    out_hbm.at[pl.ds(base + (c0 + i) * _CHUNK, _CHUNK)],
                    ssems[i],
                )
            return carry

        lax.fori_loop(0, n_grp, body, 0)
        for i in range(_NBUF):
            pltpu.make_async_copy(
                rows[i], out_hbm.at[pl.ds(base, _CHUNK)], ssems[i]
            ).wait()

    return gather_kernel


def kernel(x, table):
    b = x.shape[0] * x.shape[1]
    d = table.shape[1]
    n_chunks = b // (_NW * _CHUNK)
    idx = jnp.reshape(x.astype(jnp.int32), (_NW, n_chunks, _CHUNK))
    tpad = jnp.pad(table, ((0, 0), (0, _DPAD - d)))
    out = _make_gather(n_chunks, d)(idx, tpad)
    return jnp.reshape(out[:, :d], (x.shape[0], x.shape[1], d))
